# baseline (speedup 1.0000x reference)
import functools

import jax
import jax.numpy as jnp
from jax import lax
from jax.experimental import pallas as pl
from jax.experimental.pallas import tpu as pltpu
from jax.experimental.pallas import tpu_sc as plsc

_T = 20
_L = 8
_D = 64
_BT = 4096
_G = 32
_GW = _G // 2


def _sc_gather_body(idx_hbm, table_hbm, out_hbm, idx_v, rows_v, sem):
    wid = lax.axis_index("c") * 16 + lax.axis_index("s")

    @pl.when(wid == 0)
    def _():
        pltpu.sync_copy(idx_hbm, idx_v)
        pltpu.async_copy(table_hbm.at[idx_v], rows_v, sem).wait()
        pltpu.sync_copy(rows_v, out_hbm)


def _gather_rows(step_ids, table_wide):
    mesh = plsc.VectorSubcoreMesh(core_axis_name="c", subcore_axis_name="s")
    k = functools.partial(
        pl.kernel,
        out_type=jax.ShapeDtypeStruct((_GW, 2 * _D), jnp.float32),
        mesh=mesh,
        scratch_types=[
            pltpu.VMEM((_GW,), jnp.int32),
            pltpu.VMEM((_GW, 2 * _D), jnp.float32),
            pltpu.SemaphoreType.DMA,
        ],
    )(_sc_gather_body)
    return k(step_ids, table_wide)


def _tc_body(prog_ref, s_ref, wcat_ref, emb_ref, out_ref, tr_ref):
    prog = prog_ref[...]
    sel = jax.nn.softmax(prog, axis=-1)
    wcat = wcat_ref[...]
    bias = jnp.dot(emb_ref[0:_T, :], wcat,
                   preferred_element_type=jnp.float32)
    eye = (jax.lax.broadcasted_iota(jnp.int32, (_D, _D), 0)
           == jax.lax.broadcasted_iota(jnp.int32, (_D, _D), 1)
           ).astype(jnp.float32)
    cur = s_ref[...]
    for i in range(_T):
        y = jnp.tanh(
            jnp.dot(cur, wcat, preferred_element_type=jnp.float32)
            + bias[i:i + 1, :])
        m = jnp.concatenate(
            [eye * sel[i:i + 1, l:l + 1] for l in range(_L)], axis=0)
        cur = jnp.dot(y, m, preferred_element_type=jnp.float32)
    out_ref[...] = cur
    tr_ref[...] = prog


def kernel(s, prog, lib_W, emb_table):
    B = s.shape[0]
    step_ids = jnp.arange(_GW, dtype=jnp.int32)
    table_wide = emb_table.reshape(emb_table.shape[0] // 2, 2 * _D)
    rows = _gather_rows(step_ids, table_wide).reshape(_G, _D)
    wcat = jnp.transpose(lib_W, (1, 0, 2)).reshape(_D, _L * _D)
    grid = (B // _BT,)
    out, trace = pl.pallas_call(
        _tc_body,
        grid=grid,
        in_specs=[
            pl.BlockSpec((_T, _L), lambda t: (0, 0)),
            pl.BlockSpec((_BT, _D), lambda t: (t, 0)),
            pl.BlockSpec((_D, _L * _D), lambda t: (0, 0)),
            pl.BlockSpec((_G, _D), lambda t: (0, 0)),
        ],
        out_specs=[
            pl.BlockSpec((_BT, _D), lambda t: (t, 0)),
            pl.BlockSpec((_T, _L), lambda t: (0, 0)),
        ],
        out_shape=[
            jax.ShapeDtypeStruct((B, _D), jnp.float32),
            jax.ShapeDtypeStruct((_T, _L), jnp.float32),
        ],
        compiler_params=pltpu.CompilerParams(
            dimension_semantics=("arbitrary",),
        ),
    )(prog, s, wcat, rows)
    return (out, trace)


# speedup vs baseline: 1.1965x; 1.1965x over previous
import functools

import jax
import jax.numpy as jnp
from jax import lax
from jax.experimental import pallas as pl
from jax.experimental.pallas import tpu as pltpu
from jax.experimental.pallas import tpu_sc as plsc

_T = 20
_L = 8
_D = 64
_BT = 1024
_G = 32
_GW = _G // 2


def _sc_gather_body(idx_hbm, table_hbm, out_hbm, idx_v, rows_v, sem):
    wid = lax.axis_index("c") * 16 + lax.axis_index("s")

    @pl.when(wid == 0)
    def _():
        pltpu.sync_copy(idx_hbm, idx_v)
        pltpu.async_copy(table_hbm.at[idx_v], rows_v, sem).wait()
        pltpu.sync_copy(rows_v, out_hbm)


def _gather_rows(step_ids, table_wide):
    mesh = plsc.VectorSubcoreMesh(core_axis_name="c", subcore_axis_name="s")
    k = functools.partial(
        pl.kernel,
        out_type=jax.ShapeDtypeStruct((_GW, 2 * _D), jnp.float32),
        mesh=mesh,
        scratch_types=[
            pltpu.VMEM((_GW,), jnp.int32),
            pltpu.VMEM((_GW, 2 * _D), jnp.float32),
            pltpu.SemaphoreType.DMA,
        ],
    )(_sc_gather_body)
    return k(step_ids, table_wide)


def _tc_body(prog_ref, s_ref, wcat_ref, emb_ref, out_ref, tr_ref):
    prog = prog_ref[...]
    sel = jax.nn.softmax(prog, axis=-1)
    wcat = wcat_ref[...]
    bias = jnp.dot(emb_ref[0:_T, :], wcat,
                   preferred_element_type=jnp.float32)
    eye = (jax.lax.broadcasted_iota(jnp.int32, (_D, _D), 0)
           == jax.lax.broadcasted_iota(jnp.int32, (_D, _D), 1)
           ).astype(jnp.float32)
    cur = s_ref[...]
    for i in range(_T):
        y = jnp.tanh(
            jnp.dot(cur, wcat, preferred_element_type=jnp.float32)
            + bias[i:i + 1, :])
        m = jnp.concatenate(
            [eye * sel[i:i + 1, l:l + 1] for l in range(_L)], axis=0)
        cur = jnp.dot(y, m, preferred_element_type=jnp.float32)
    out_ref[...] = cur
    tr_ref[...] = prog


def kernel(s, prog, lib_W, emb_table):
    B = s.shape[0]
    step_ids = jnp.arange(_GW, dtype=jnp.int32)
    table_wide = emb_table.reshape(emb_table.shape[0] // 2, 2 * _D)
    rows = _gather_rows(step_ids, table_wide).reshape(_G, _D)
    wcat = jnp.transpose(lib_W, (1, 0, 2)).reshape(_D, _L * _D)
    grid = (B // _BT,)
    out, trace = pl.pallas_call(
        _tc_body,
        grid=grid,
        in_specs=[
            pl.BlockSpec((_T, _L), lambda t: (0, 0)),
            pl.BlockSpec((_BT, _D), lambda t: (t, 0)),
            pl.BlockSpec((_D, _L * _D), lambda t: (0, 0)),
            pl.BlockSpec((_G, _D), lambda t: (0, 0)),
        ],
        out_specs=[
            pl.BlockSpec((_BT, _D), lambda t: (t, 0)),
            pl.BlockSpec((_T, _L), lambda t: (0, 0)),
        ],
        out_shape=[
            jax.ShapeDtypeStruct((B, _D), jnp.float32),
            jax.ShapeDtypeStruct((_T, _L), jnp.float32),
        ],
        compiler_params=pltpu.CompilerParams(
            dimension_semantics=("arbitrary",),
        ),
    )(prog, s, wcat, rows)
    return (out, trace)


# speedup vs baseline: 1.6469x; 1.3765x over previous
import functools

import jax
import jax.numpy as jnp
from jax import lax
from jax.experimental import pallas as pl
from jax.experimental.pallas import tpu as pltpu
from jax.experimental.pallas import tpu_sc as plsc

_T = 20
_L = 8
_D = 64
_BT = 2048
_G = 32
_GW = _G // 2


def _sc_gather_body(idx_hbm, table_hbm, out_hbm, idx_v, rows_v, sem):
    wid = lax.axis_index("c") * 16 + lax.axis_index("s")

    @pl.when(wid == 0)
    def _():
        pltpu.sync_copy(idx_hbm, idx_v)
        pltpu.async_copy(table_hbm.at[idx_v], rows_v, sem).wait()
        pltpu.sync_copy(rows_v, out_hbm)


def _gather_rows(step_ids, table_wide):
    mesh = plsc.VectorSubcoreMesh(core_axis_name="c", subcore_axis_name="s")
    k = functools.partial(
        pl.kernel,
        out_type=jax.ShapeDtypeStruct((_GW, 2 * _D), jnp.float32),
        mesh=mesh,
        scratch_types=[
            pltpu.VMEM((_GW,), jnp.int32),
            pltpu.VMEM((_GW, 2 * _D), jnp.float32),
            pltpu.SemaphoreType.DMA,
        ],
    )(_sc_gather_body)
    return k(step_ids, table_wide)


def _tc_body(prog_ref, s_ref, wcat_ref, emb_ref, out_ref, tr_ref):
    prog = prog_ref[...]
    sel = jax.nn.softmax(prog, axis=-1)
    wcat = wcat_ref[...]
    bias = jnp.dot(emb_ref[0:_T, :], wcat,
                   preferred_element_type=jnp.float32)
    lane_l = jax.lax.broadcasted_iota(jnp.int32, (1, _L * _D), 1) // _D
    selw = jnp.zeros((_T, _L * _D), dtype=jnp.float32)
    for l in range(_L):
        selw = jnp.where(lane_l == l, sel[:, l:l + 1], selw)
    cur = s_ref[...]
    for i in range(_T):
        y = jnp.tanh(
            jnp.dot(cur, wcat, preferred_element_type=jnp.float32)
            + bias[i:i + 1, :])
        yw = y * selw[i:i + 1, :]
        a = yw[:, 0:256] + yw[:, 256:512]
        a = a[:, 0:128] + a[:, 128:256]
        cur = a[:, 0:64] + a[:, 64:128]
    out_ref[...] = cur
    tr_ref[...] = prog


def kernel(s, prog, lib_W, emb_table):
    B = s.shape[0]
    step_ids = jnp.arange(_GW, dtype=jnp.int32)
    table_wide = emb_table.reshape(emb_table.shape[0] // 2, 2 * _D)
    rows = _gather_rows(step_ids, table_wide).reshape(_G, _D)
    wcat = jnp.transpose(lib_W, (1, 0, 2)).reshape(_D, _L * _D)
    grid = (B // _BT,)
    out, trace = pl.pallas_call(
        _tc_body,
        grid=grid,
        in_specs=[
            pl.BlockSpec((_T, _L), lambda t: (0, 0)),
            pl.BlockSpec((_BT, _D), lambda t: (t, 0)),
            pl.BlockSpec((_D, _L * _D), lambda t: (0, 0)),
            pl.BlockSpec((_G, _D), lambda t: (0, 0)),
        ],
        out_specs=[
            pl.BlockSpec((_BT, _D), lambda t: (t, 0)),
            pl.BlockSpec((_T, _L), lambda t: (0, 0)),
        ],
        out_shape=[
            jax.ShapeDtypeStruct((B, _D), jnp.float32),
            jax.ShapeDtypeStruct((_T, _L), jnp.float32),
        ],
        compiler_params=pltpu.CompilerParams(
            dimension_semantics=("arbitrary",),
        ),
    )(prog, s, wcat, rows)
    return (out, trace)


# speedup vs baseline: 1.9367x; 1.1760x over previous
import functools

import jax
import jax.numpy as jnp
from jax import lax
from jax.experimental import pallas as pl
from jax.experimental.pallas import tpu as pltpu
from jax.experimental.pallas import tpu_sc as plsc

_T = 20
_L = 8
_D = 64
_BT = 2048
_G = 32
_GW = _G // 2


def _sc_gather_body(table_hbm, out_hbm, rows_v):
    wid = lax.axis_index("c") * 16 + lax.axis_index("s")

    @pl.when(wid == 0)
    def _():
        pltpu.sync_copy(table_hbm.at[pl.ds(0, _G)], rows_v)
        pltpu.sync_copy(rows_v, out_hbm)


def _gather_rows(emb_table):
    mesh = plsc.VectorSubcoreMesh(core_axis_name="c", subcore_axis_name="s")
    k = functools.partial(
        pl.kernel,
        out_type=jax.ShapeDtypeStruct((_G, _D), jnp.float32),
        mesh=mesh,
        scratch_types=[
            pltpu.VMEM((_G, _D), jnp.float32),
        ],
    )(_sc_gather_body)
    return k(emb_table)


def _tc_body(prog_ref, s_ref, wcat_ref, emb_ref, out_ref, tr_ref):
    prog = prog_ref[...]
    sel = jax.nn.softmax(prog, axis=-1)
    wcat = wcat_ref[...]
    bias = jnp.dot(emb_ref[0:_T, :], wcat,
                   preferred_element_type=jnp.float32)
    lane_l = jax.lax.broadcasted_iota(jnp.int32, (1, _L * _D), 1) // _D
    selw = jnp.zeros((_T, _L * _D), dtype=jnp.float32)
    for l in range(_L):
        selw = jnp.where(lane_l == l, sel[:, l:l + 1], selw)
    cur = s_ref[...]
    for i in range(_T):
        y = jnp.tanh(
            jnp.dot(cur, wcat, preferred_element_type=jnp.float32)
            + bias[i:i + 1, :])
        yw = y * selw[i:i + 1, :]
        a = yw[:, 0:256] + yw[:, 256:512]
        a = a[:, 0:128] + a[:, 128:256]
        cur = a[:, 0:64] + a[:, 64:128]
    out_ref[...] = cur
    tr_ref[...] = prog


def kernel(s, prog, lib_W, emb_table):
    B = s.shape[0]
    rows = _gather_rows(emb_table)
    wcat = jnp.transpose(lib_W, (1, 0, 2)).reshape(_D, _L * _D)
    grid = (B // _BT,)
    out, trace = pl.pallas_call(
        _tc_body,
        grid=grid,
        in_specs=[
            pl.BlockSpec((_T, _L), lambda t: (0, 0)),
            pl.BlockSpec((_BT, _D), lambda t: (t, 0)),
            pl.BlockSpec((_D, _L * _D), lambda t: (0, 0)),
            pl.BlockSpec((_G, _D), lambda t: (0, 0)),
        ],
        out_specs=[
            pl.BlockSpec((_BT, _D), lambda t: (t, 0)),
            pl.BlockSpec((_T, _L), lambda t: (0, 0)),
        ],
        out_shape=[
            jax.ShapeDtypeStruct((B, _D), jnp.float32),
            jax.ShapeDtypeStruct((_T, _L), jnp.float32),
        ],
        compiler_params=pltpu.CompilerParams(
            dimension_semantics=("parallel",),
        ),
    )(prog, s, wcat, rows)
    return (out, trace)


# speedup vs baseline: 1.9393x; 1.0013x over previous
import functools

import jax
import jax.numpy as jnp
from jax import lax
from jax.experimental import pallas as pl
from jax.experimental.pallas import tpu as pltpu
from jax.experimental.pallas import tpu_sc as plsc

_T = 20
_L = 8
_D = 64
_BT = 2048
_G = 32


def _sc_gather_body(table_hbm, out_hbm, rows_v):
    wid = lax.axis_index("c") * 16 + lax.axis_index("s")

    @pl.when(wid == 0)
    def _():
        pltpu.sync_copy(table_hbm.at[pl.ds(0, _G)], rows_v)
        pltpu.sync_copy(rows_v, out_hbm)


def _gather_rows(emb_table):
    mesh = plsc.VectorSubcoreMesh(core_axis_name="c", subcore_axis_name="s")
    k = functools.partial(
        pl.kernel,
        out_type=jax.ShapeDtypeStruct((_G, _D), jnp.float32),
        mesh=mesh,
        scratch_types=[
            pltpu.VMEM((_G, _D), jnp.float32),
        ],
    )(_sc_gather_body)
    return k(emb_table)


def _tc_body(prog_ref, s_ref, wcat_ref, emb_ref, out_ref, tr_ref):
    prog = prog_ref[...]
    sel = jax.nn.softmax(prog, axis=-1)
    wcat = wcat_ref[...]
    bias = jnp.dot(emb_ref[0:_T, :], wcat,
                   preferred_element_type=jnp.float32)
    lane_l = jax.lax.broadcasted_iota(jnp.int32, (1, _L * _D), 1) // _D
    selw = jnp.zeros((_T, _L * _D), dtype=jnp.float32)
    for l in range(_L):
        selw = jnp.where(lane_l == l, sel[:, l:l + 1], selw)
    cur = s_ref[...]
    for i in range(_T):
        y = jnp.tanh(
            jnp.dot(cur, wcat, preferred_element_type=jnp.float32)
            + bias[i:i + 1, :])
        yw = y * selw[i:i + 1, :]
        a = yw[:, 0:256] + yw[:, 256:512]
        a = a[:, 0:128] + a[:, 128:256]
        cur = a[:, 0:64] + a[:, 64:128]
    out_ref[...] = cur
    tr_ref[...] = prog


def kernel(s, prog, lib_W, emb_table):
    B = s.shape[0]
    rows = _gather_rows(emb_table)
    wcat = jnp.transpose(lib_W, (1, 0, 2)).reshape(_D, _L * _D)
    grid = (B // _BT,)
    out, trace = pl.pallas_call(
        _tc_body,
        grid=grid,
        in_specs=[
            pl.BlockSpec((_T, _L), lambda t: (0, 0)),
            pl.BlockSpec((_BT, _D), lambda t: (t, 0)),
            pl.BlockSpec((_D, _L * _D), lambda t: (0, 0)),
            pl.BlockSpec((_G, _D), lambda t: (0, 0)),
        ],
        out_specs=[
            pl.BlockSpec((_BT, _D), lambda t: (t, 0)),
            pl.BlockSpec((_T, _L), lambda t: (0, 0)),
        ],
        out_shape=[
            jax.ShapeDtypeStruct((B, _D), jnp.float32),
            jax.ShapeDtypeStruct((_T, _L), jnp.float32),
        ],
        compiler_params=pltpu.CompilerParams(
            dimension_semantics=("parallel",),
        ),
    )(prog, s, wcat, rows)
    return (out, trace)
